# baseline (speedup 1.0000x reference)
import jax
import jax.numpy as jnp
from jax import lax
from jax.experimental import pallas as pl
from jax.experimental.pallas import tpu as pltpu
from jax.experimental.pallas import tpu_sc as plsc

NEG_SLOPE = 0.2
NC, NS, LANES = 2, 16, 16
NW = NC * NS
CH = 128
HF = 64
NODE_PAD = 128


def _tc_project(x_pad, W, att2):
    NP, D = x_pad.shape

    def body(x_ref, w_ref, a_ref, xp_ref, a2_ref):
        xp = jnp.dot(x_ref[...], w_ref[...], preferred_element_type=jnp.float32)
        xp_ref[...] = xp
        a2_ref[...] = lax.dot_general(
            a_ref[...], xp, (((1,), (1,)), ((), ())),
            preferred_element_type=jnp.float32)

    return pl.pallas_call(
        body,
        out_shape=[
            jax.ShapeDtypeStruct((NP, D), jnp.float32),
            jax.ShapeDtypeStruct((2, NP), jnp.float32),
        ],
    )(x_pad, W, att2)


def _sc_aggregate(xpp, a_srcH, a_dstH, e3, n_nodes, n_edges, n_edges_real,
                  chunks):
    NP, DP = xpp.shape
    D = 2 * DP
    rpt = NP // NS
    CHUNKS = chunks
    EH = n_edges // HF

    def body(xpp_hbm, as_hbm, ad_hbm, e_hbm, out_hbm, den_hbm,
             sidx_v, didx_v, asv_v, adv_v, alpha_v, rows_v, stage_v, den_v,
             accum_sh, den_sh, sem_i, sem_l, sem_g, sem_sr, sem_sd):
        cid = lax.axis_index("c")
        sid = lax.axis_index("s")
        wid = cid * NS + sid

        iota16 = lax.iota(jnp.int32, LANES)
        zeros16 = jnp.zeros((LANES,), jnp.float32)

        def synth_idx(jj, slot):
            tbase = (wid * CHUNKS + jj) * CH - n_edges
            for h in range(2):
                for i in range(CH // LANES // 2):
                    t = tbase + h * HF + i * LANES + iota16
                    v = jnp.where(t < n_nodes, t, t - n_nodes)
                    sidx_v[slot, h, pl.ds(i * LANES, LANES)] = v
                    didx_v[slot, h, pl.ds(i * LANES, LANES)] = v

        def prep_idx(jj, slot, sync):
            b2 = (wid * CHUNKS + jj) * 2

            @pl.when(b2 < EH)
            def _():
                if sync:
                    pltpu.sync_copy(e_hbm.at[0, pl.ds(b2, 2)], sidx_v.at[slot])
                    pltpu.sync_copy(e_hbm.at[1, pl.ds(b2, 2)], didx_v.at[slot])
                else:
                    pltpu.async_copy(e_hbm.at[0, pl.ds(b2, 2)],
                                     sidx_v.at[slot], sem_i.at[slot])
                    pltpu.async_copy(e_hbm.at[1, pl.ds(b2, 2)],
                                     didx_v.at[slot], sem_i.at[slot])

            @pl.when(b2 >= EH)
            def _():
                synth_idx(jj, slot)

        def wait_idx(jj, slot):
            b2 = (wid * CHUNKS + jj) * 2

            @pl.when(b2 < EH)
            def _():
                pltpu.make_async_copy(e_hbm.at[0, pl.ds(0, 2)],
                                      sidx_v.at[slot], sem_i.at[slot]).wait()
                pltpu.make_async_copy(e_hbm.at[1, pl.ds(0, 2)],
                                      didx_v.at[slot], sem_i.at[slot]).wait()

        def issue_logit(islot, b):
            for h in range(2):
                pltpu.async_copy(as_hbm.at[sidx_v.at[islot, h]],
                                 asv_v.at[b, h], sem_l.at[b])
                pltpu.async_copy(ad_hbm.at[didx_v.at[islot, h]],
                                 adv_v.at[b, h], sem_l.at[b])

        def wait_logit(b):
            for h in range(2):
                pltpu.make_async_copy(as_hbm.at[pl.ds(0, HF)],
                                      asv_v.at[b, h], sem_l.at[b]).wait()
                pltpu.make_async_copy(ad_hbm.at[pl.ds(0, HF)],
                                      adv_v.at[b, h], sem_l.at[b]).wait()

        def issue_rows(islot, b):
            for h in range(2):
                pltpu.async_copy(xpp_hbm.at[sidx_v.at[islot, h]],
                                 rows_v.at[b, h], sem_g.at[b])

        def wait_rows(b):
            for h in range(2):
                pltpu.make_async_copy(xpp_hbm.at[pl.ds(0, HF)],
                                      rows_v.at[b, h], sem_g.at[b]).wait()

        def issue_scatter(islot, b, h):
            pltpu.async_copy(stage_v.at[h], accum_sh.at[didx_v.at[islot, h]],
                             sem_sr.at[b], add=True)
            pltpu.async_copy(alpha_v.at[b, h], den_sh.at[didx_v.at[islot, h]],
                             sem_sd.at[b], add=True)

        def wait_scatter(b):
            for h in range(2):
                pltpu.make_async_copy(out_hbm.at[0, pl.ds(0, HF)],
                                      stage_v.at[h], sem_sr.at[b]).wait()
                pltpu.make_async_copy(as_hbm.at[pl.ds(0, HF)],
                                      alpha_v.at[b, h], sem_sd.at[b]).wait()

        def zrow(r, _):
            for c in range(D // LANES):
                stage_v[0, r, pl.ds(c * LANES, LANES)] = zeros16
            return 0

        lax.fori_loop(0, HF, zrow, 0)
        for i in range(HF // LANES):
            alpha_v[0, 0, pl.ds(i * LANES, LANES)] = zeros16
        for off in range(0, rpt, HF):
            cnt = min(HF, rpt - off)
            pltpu.sync_copy(stage_v.at[0, pl.ds(0, cnt)],
                            accum_sh.at[pl.ds(sid * rpt + off, cnt)])
            pltpu.sync_copy(alpha_v.at[0, 0, pl.ds(0, cnt)],
                            den_sh.at[pl.ds(sid * rpt + off, cnt)])
        plsc.subcore_barrier()

        prep_idx(0, 0, sync=True)
        prep_idx(1, 1, sync=False)
        issue_logit(0, 0)
        issue_rows(0, 0)

        def chunk_body(j, _):
            b = lax.rem(j, 2)
            nb = 1 - b
            i3 = lax.rem(j, 3)
            i3n = lax.rem(j + 1, 3)
            i3nn = lax.rem(j + 2, 3)

            @pl.when(j >= 1)
            def _():
                wait_scatter(nb)

            @pl.when(j <= CHUNKS - 2)
            def _():
                wait_idx(j + 1, i3n)
                issue_logit(i3n, nb)
                issue_rows(i3n, nb)

            @pl.when(j <= CHUNKS - 3)
            def _():
                prep_idx(j + 2, i3nn, sync=False)

            wait_rows(b)
            wait_logit(b)

            base = (wid * CHUNKS + j) * CH
            for h in range(2):
                for i in range(HF // LANES):
                    al = (asv_v[b, h, pl.ds(i * LANES, LANES)]
                          + adv_v[b, h, pl.ds(i * LANES, LANES)])
                    al = jnp.where(al > 0, al, NEG_SLOPE * al)
                    ev = jnp.exp(al)
                    eid = base + h * HF + i * LANES + iota16
                    ev = jnp.where(eid < n_edges_real, ev, 0.0)
                    alpha_v[b, h, pl.ds(i * LANES, LANES)] = ev

            for h in range(2):
                hvec = jnp.full((LANES,), h, jnp.int32)
                bvec = jnp.broadcast_to(b, (LANES,)).astype(jnp.int32)

                def srow(r, _, h=h, hvec=hvec, bvec=bvec):
                    asp = plsc.load_gather(
                        alpha_v,
                        [bvec, hvec,
                         jnp.broadcast_to(r, (LANES,)).astype(jnp.int32)])
                    for c in range(DP // LANES):
                        w = rows_v[b, h, r, pl.ds(c * LANES, LANES)]
                        wb = plsc.bitcast(w, jnp.bfloat16)
                        lo, hi = plsc.unpack(
                            wb, format=plsc.PackFormat.INTERLEAVED)
                        stage_v[h, r, pl.ds(c * LANES, LANES)] = lo * asp
                        stage_v[h, r, pl.ds(DP + c * LANES, LANES)] = hi * asp
                    return 0

                lax.fori_loop(0, HF, srow, 0)
                issue_scatter(i3, b, h)
            return 0

        lax.fori_loop(0, CHUNKS, chunk_body, 0)
        wait_scatter((CHUNKS - 1) % 2)
        plsc.subcore_barrier()

        for off in range(0, rpt, HF):
            cnt = min(HF, rpt - off)
            pltpu.sync_copy(accum_sh.at[pl.ds(sid * rpt + off, cnt)],
                            stage_v.at[0, pl.ds(0, cnt)])
            pltpu.sync_copy(stage_v.at[0, pl.ds(0, cnt)],
                            out_hbm.at[cid, pl.ds(sid * rpt + off, cnt)])
        pltpu.sync_copy(den_sh.at[pl.ds(sid * rpt, rpt)], den_v)
        pltpu.sync_copy(den_v, den_hbm.at[pl.ds(cid * NP + sid * rpt, rpt)])

    mesh = plsc.VectorSubcoreMesh(core_axis_name="c", subcore_axis_name="s")
    return pl.kernel(
        body,
        out_type=[
            jax.ShapeDtypeStruct((NC, NP, D), jnp.float32),
            jax.ShapeDtypeStruct((NC * NP,), jnp.float32),
        ],
        mesh=mesh,
        compiler_params=pltpu.CompilerParams(needs_layout_passes=False,
                                             use_tc_tiling_on_sc=False),
        scratch_types=[
            pltpu.VMEM((3, 2, HF), jnp.int32),
            pltpu.VMEM((3, 2, HF), jnp.int32),
            pltpu.VMEM((2, 2, HF), jnp.float32),
            pltpu.VMEM((2, 2, HF), jnp.float32),
            pltpu.VMEM((2, 2, HF), jnp.float32),
            pltpu.VMEM((2, 2, HF, DP), jnp.float32),
            pltpu.VMEM((2, HF, D), jnp.float32),
            pltpu.VMEM((rpt,), jnp.float32),
            pltpu.VMEM_SHARED((NP, D), jnp.float32),
            pltpu.VMEM_SHARED((NP,), jnp.float32),
            pltpu.SemaphoreType.DMA((3,)),
            pltpu.SemaphoreType.DMA((2,)),
            pltpu.SemaphoreType.DMA((2,)),
            pltpu.SemaphoreType.DMA((2,)),
            pltpu.SemaphoreType.DMA((2,)),
        ],
    )(xpp, a_srcH, a_dstH, e3)


def _tc_combine(accd, dend3, bias2, n_real):
    D = bias2.shape[1]

    def body(acc_ref, den_ref, b_ref, o_ref):
        num = acc_ref[0] + acc_ref[1]
        den = den_ref[0] + den_ref[1]
        o_ref[...] = (num[:n_real] / (den[:n_real] + 1e-16)) + b_ref[...]

    return pl.pallas_call(
        body,
        out_shape=jax.ShapeDtypeStruct((n_real, D), jnp.float32),
    )(accd, dend3, bias2)


def kernel(x, edge_index, W, att_src, att_dst, bias):
    n, d_in = x.shape
    e = edge_index.shape[1]

    npad = -n % NODE_PAD
    NP = n + npad
    x_pad = jnp.concatenate([x, jnp.zeros((npad, d_in), x.dtype)], axis=0)
    att2 = jnp.stack([att_src, att_dst])

    xp, a2 = _tc_project(x_pad, W, att2)
    xpb = xp.astype(jnp.bfloat16)
    xpp = lax.bitcast_convert_type(
        jnp.stack([xpb[:, :d_in // 2], xpb[:, d_in // 2:]], axis=-1),
        jnp.float32)

    ee = e + n
    chunks = -(-ee // (NW * CH))
    e3 = edge_index.reshape(2, e // HF, HF)

    accd, dend = _sc_aggregate(xpp, a2[0], a2[1], e3, n, e, ee, chunks)
    dend3 = dend.reshape(NC, NP, 1)
    return _tc_combine(accd, dend3, bias[None, :], n)


# speedup vs baseline: 1.3817x; 1.3817x over previous
import jax
import jax.numpy as jnp
from jax import lax
from jax.experimental import pallas as pl
from jax.experimental.pallas import tpu as pltpu
from jax.experimental.pallas import tpu_sc as plsc

NEG_SLOPE = 0.2
NC, NS, LANES = 2, 16, 16
NW = NC * NS
CHUNK = 128
NODE_PAD = 128


def _tc_project(x_pad, W, att2):
    NP, D = x_pad.shape

    def body(x_ref, w_ref, a_ref, xp_ref, a2_ref):
        xp = jnp.dot(x_ref[...], w_ref[...], preferred_element_type=jnp.float32)
        xp_ref[...] = xp
        a2_ref[...] = lax.dot_general(
            a_ref[...], xp, (((1,), (1,)), ((), ())),
            preferred_element_type=jnp.float32)

    return pl.pallas_call(
        body,
        out_shape=[
            jax.ShapeDtypeStruct((NP, D), jnp.float32),
            jax.ShapeDtypeStruct((2, NP), jnp.float32),
        ],
    )(x_pad, W, att2)


def _sc_aggregate(xp, a_srcH, a_dstH, src3, dst3, n_edges_real):
    NP, D = xp.shape
    CHUNKS = src3.shape[1]
    rpt = NP // NS

    def body(xp_hbm, as_hbm, ad_hbm, src_hbm, dst_hbm, out_hbm, den_hbm,
             sidx_v, didx_v, asv_v, adv_v, alpha_v, rows_v, den_v,
             accum_sh, den_sh, sem_i, sem_l, sem_g, sem_sr, sem_sd):
        cid = lax.axis_index("c")
        sid = lax.axis_index("s")
        wid = cid * NS + sid

        zeros16 = jnp.zeros((LANES,), jnp.float32)

        def zrow(r, _):
            for c in range(D // LANES):
                rows_v[0, r, pl.ds(c * LANES, LANES)] = zeros16
            return 0

        lax.fori_loop(0, CHUNK, zrow, 0)
        for i in range(CHUNK // LANES):
            alpha_v[0, pl.ds(i * LANES, LANES)] = zeros16
        for off in range(0, rpt, CHUNK):
            cnt = min(CHUNK, rpt - off)
            pltpu.sync_copy(rows_v.at[0, pl.ds(0, cnt)],
                            accum_sh.at[pl.ds(sid * rpt + off, cnt)])
            pltpu.sync_copy(alpha_v.at[0, pl.ds(0, cnt)],
                            den_sh.at[pl.ds(sid * rpt + off, cnt)])
        plsc.subcore_barrier()

        iota16 = lax.iota(jnp.int32, LANES)

        def issue_idx(j, slot):
            pltpu.async_copy(src_hbm.at[wid, j], sidx_v.at[slot], sem_i.at[slot])
            pltpu.async_copy(dst_hbm.at[wid, j], didx_v.at[slot], sem_i.at[slot])

        def wait_idx(slot):
            pltpu.make_async_copy(src_hbm.at[wid, 0], sidx_v.at[slot],
                                  sem_i.at[slot]).wait()
            pltpu.make_async_copy(dst_hbm.at[wid, 0], didx_v.at[slot],
                                  sem_i.at[slot]).wait()

        def issue_logit(islot, slot):
            pltpu.async_copy(as_hbm.at[sidx_v.at[islot]], asv_v.at[slot],
                             sem_l.at[slot])
            pltpu.async_copy(ad_hbm.at[didx_v.at[islot]], adv_v.at[slot],
                             sem_l.at[slot])

        def wait_logit(slot):
            pltpu.make_async_copy(as_hbm.at[pl.ds(0, CHUNK)], asv_v.at[slot],
                                  sem_l.at[slot]).wait()
            pltpu.make_async_copy(ad_hbm.at[pl.ds(0, CHUNK)], adv_v.at[slot],
                                  sem_l.at[slot]).wait()

        def issue_rows(islot, slot):
            pltpu.async_copy(xp_hbm.at[sidx_v.at[islot]], rows_v.at[slot],
                             sem_g.at[slot])

        def wait_rows(slot):
            pltpu.make_async_copy(xp_hbm.at[pl.ds(0, CHUNK)], rows_v.at[slot],
                                  sem_g.at[slot]).wait()

        def issue_scatter(islot, slot):
            pltpu.async_copy(rows_v.at[slot], accum_sh.at[didx_v.at[islot]],
                             sem_sr.at[slot], add=True)
            pltpu.async_copy(alpha_v.at[slot], den_sh.at[didx_v.at[islot]],
                             sem_sd.at[slot], add=True)

        def wait_scatter(slot):
            pltpu.make_async_copy(xp_hbm.at[pl.ds(0, CHUNK)], rows_v.at[slot],
                                  sem_sr.at[slot]).wait()
            pltpu.make_async_copy(as_hbm.at[pl.ds(0, CHUNK)], alpha_v.at[slot],
                                  sem_sd.at[slot]).wait()

        pltpu.sync_copy(src_hbm.at[wid, 0], sidx_v.at[0])
        pltpu.sync_copy(dst_hbm.at[wid, 0], didx_v.at[0])
        issue_idx(1, 1)
        issue_logit(0, 0)
        issue_rows(0, 0)

        def chunk_body(j, _):
            b = lax.rem(j, 2)
            nb = 1 - b
            i3 = lax.rem(j, 3)
            i3n = lax.rem(j + 1, 3)
            i3nn = lax.rem(j + 2, 3)

            @pl.when(j >= 1)
            def _():
                wait_scatter(nb)

            @pl.when(j <= CHUNKS - 2)
            def _():
                wait_idx(i3n)
                issue_logit(i3n, nb)
                issue_rows(i3n, nb)

            wait_rows(b)
            wait_logit(b)

            @pl.when(j <= CHUNKS - 3)
            def _():
                issue_idx(j + 2, i3nn)

            base = (wid * CHUNKS + j) * CHUNK
            for i in range(CHUNK // LANES):
                al = (asv_v[b, pl.ds(i * LANES, LANES)]
                      + adv_v[b, pl.ds(i * LANES, LANES)])
                al = jnp.where(al > 0, al, NEG_SLOPE * al)
                ev = jnp.exp(al)
                eid = base + i * LANES + iota16
                ev = jnp.where(eid < n_edges_real, ev, 0.0)
                alpha_v[b, pl.ds(i * LANES, LANES)] = ev

            def srow(r, _):
                asp = plsc.load_gather(
                    alpha_v,
                    [jnp.broadcast_to(b, (LANES,)).astype(jnp.int32),
                     jnp.broadcast_to(r, (LANES,)).astype(jnp.int32)])
                for c in range(D // LANES):
                    rows_v[b, r, pl.ds(c * LANES, LANES)] = (
                        rows_v[b, r, pl.ds(c * LANES, LANES)] * asp)
                return 0

            lax.fori_loop(0, CHUNK, srow, 0)
            issue_scatter(i3, b)
            return 0

        lax.fori_loop(0, CHUNKS, chunk_body, 0)
        wait_scatter((CHUNKS - 1) % 2)
        plsc.subcore_barrier()

        for off in range(0, rpt, CHUNK):
            cnt = min(CHUNK, rpt - off)
            pltpu.sync_copy(accum_sh.at[pl.ds(sid * rpt + off, cnt)],
                            rows_v.at[0, pl.ds(0, cnt)])
            pltpu.sync_copy(rows_v.at[0, pl.ds(0, cnt)],
                            out_hbm.at[cid, pl.ds(sid * rpt + off, cnt)])
        pltpu.sync_copy(den_sh.at[pl.ds(sid * rpt, rpt)], den_v)
        pltpu.sync_copy(den_v, den_hbm.at[pl.ds(cid * NP + sid * rpt, rpt)])

    mesh = plsc.VectorSubcoreMesh(core_axis_name="c", subcore_axis_name="s")
    return pl.kernel(
        body,
        out_type=[
            jax.ShapeDtypeStruct((NC, NP, D), jnp.float32),
            jax.ShapeDtypeStruct((NC * NP,), jnp.float32),
        ],
        mesh=mesh,
        compiler_params=pltpu.CompilerParams(needs_layout_passes=False,
                                             use_tc_tiling_on_sc=False),
        scratch_types=[
            pltpu.VMEM((3, CHUNK), jnp.int32),
            pltpu.VMEM((3, CHUNK), jnp.int32),
            pltpu.VMEM((2, CHUNK), jnp.float32),
            pltpu.VMEM((2, CHUNK), jnp.float32),
            pltpu.VMEM((2, CHUNK), jnp.float32),
            pltpu.VMEM((2, CHUNK, D), jnp.float32),
            pltpu.VMEM((rpt,), jnp.float32),
            pltpu.VMEM_SHARED((NP, D), jnp.float32),
            pltpu.VMEM_SHARED((NP,), jnp.float32),
            pltpu.SemaphoreType.DMA((3,)),
            pltpu.SemaphoreType.DMA((2,)),
            pltpu.SemaphoreType.DMA((2,)),
            pltpu.SemaphoreType.DMA((2,)),
            pltpu.SemaphoreType.DMA((2,)),
        ],
    )(xp, a_srcH, a_dstH, src3, dst3)


def _tc_combine(accd, dend3, bias2, n_real):
    D = bias2.shape[1]

    def body(acc_ref, den_ref, b_ref, o_ref):
        num = acc_ref[0] + acc_ref[1]
        den = den_ref[0] + den_ref[1]
        o_ref[...] = (num[:n_real] / (den[:n_real] + 1e-16)) + b_ref[...]

    return pl.pallas_call(
        body,
        out_shape=jax.ShapeDtypeStruct((n_real, D), jnp.float32),
    )(accd, dend3, bias2)


def kernel(x, edge_index, W, att_src, att_dst, bias):
    n, d_in = x.shape
    e = edge_index.shape[1]

    npad = -n % NODE_PAD
    NP = n + npad
    x_pad = jnp.concatenate([x, jnp.zeros((npad, d_in), x.dtype)], axis=0)
    att2 = jnp.stack([att_src, att_dst])

    xp, a2 = _tc_project(x_pad, W, att2)

    ee = e + n
    epad = -ee % (NW * CHUNK)
    chunks = (ee + epad) // (NW * CHUNK)
    loop = jnp.arange(n, dtype=jnp.int32)
    fill = jnp.arange(epad, dtype=jnp.int32) % NP
    src3 = jnp.concatenate([edge_index[0], loop, fill]).reshape(NW, chunks, CHUNK)
    dst3 = jnp.concatenate([edge_index[1], loop, fill]).reshape(NW, chunks, CHUNK)

    accd, dend = _sc_aggregate(xp, a2[0], a2[1], src3, dst3, ee)
    dend3 = dend.reshape(NC, NP, 1)
    return _tc_combine(accd, dend3, bias[None, :], n)


# speedup vs baseline: 1.5507x; 1.1223x over previous
import jax
import jax.numpy as jnp
from jax import lax
from jax.experimental import pallas as pl
from jax.experimental.pallas import tpu as pltpu
from jax.experimental.pallas import tpu_sc as plsc

NEG_SLOPE = 0.2
NC, NS, LANES = 2, 16, 16
NW = NC * NS
CHUNK = 128
NODE_PAD = 128


def _tc_project(x, W, att2, NP):
    n, D = x.shape

    def body(x_ref, w_ref, a_ref, xp_ref, a2_ref):
        xp = jnp.dot(x_ref[...], w_ref[...], preferred_element_type=jnp.float32)
        xp_ref[:n] = xp
        xp_ref[n:] = jnp.zeros((NP - n, D), jnp.float32)
        a2_ref[:, :n] = lax.dot_general(
            a_ref[...], xp, (((1,), (1,)), ((), ())),
            preferred_element_type=jnp.float32)
        a2_ref[:, n:] = jnp.zeros((2, NP - n), jnp.float32)

    return pl.pallas_call(
        body,
        out_shape=[
            jax.ShapeDtypeStruct((NP, D), jnp.float32),
            jax.ShapeDtypeStruct((2, NP), jnp.float32),
        ],
    )(x, W, att2)


def _sc_aggregate(xp, a_srcH, a_dstH, e3, n_nodes, n_edges, n_edges_real,
                  chunks):
    NP, D = xp.shape
    CHUNKS = chunks
    EC = n_edges // CHUNK
    rpt = NP // NS

    def body(xp_hbm, as_hbm, ad_hbm, e_hbm, out_hbm, den_hbm,
             sidx_v, didx_v, asv_v, adv_v, alpha_v, rows_v, den_v,
             accum_sh, den_sh, sem_i, sem_l, sem_g, sem_sr, sem_sd):
        cid = lax.axis_index("c")
        sid = lax.axis_index("s")
        wid = cid * NS + sid

        zeros16 = jnp.zeros((LANES,), jnp.float32)

        def zrow(r, _):
            for c in range(D // LANES):
                rows_v[0, r, pl.ds(c * LANES, LANES)] = zeros16
            return 0

        lax.fori_loop(0, CHUNK, zrow, 0)
        for i in range(CHUNK // LANES):
            alpha_v[0, pl.ds(i * LANES, LANES)] = zeros16
        for off in range(0, rpt, CHUNK):
            cnt = min(CHUNK, rpt - off)
            pltpu.sync_copy(rows_v.at[0, pl.ds(0, cnt)],
                            accum_sh.at[pl.ds(sid * rpt + off, cnt)])
            pltpu.sync_copy(alpha_v.at[0, pl.ds(0, cnt)],
                            den_sh.at[pl.ds(sid * rpt + off, cnt)])
        plsc.subcore_barrier()

        iota16 = lax.iota(jnp.int32, LANES)

        def synth_idx(jj, slot):
            tbase = (wid * CHUNKS + jj) * CHUNK - n_edges
            for i in range(CHUNK // LANES):
                t = tbase + i * LANES + iota16
                v = jnp.where(t < n_nodes, t, t - n_nodes)
                sidx_v[slot, pl.ds(i * LANES, LANES)] = v
                didx_v[slot, pl.ds(i * LANES, LANES)] = v

        def prep_idx(jj, slot, sync):
            g = wid * CHUNKS + jj

            @pl.when(g < EC)
            def _():
                if sync:
                    pltpu.sync_copy(e_hbm.at[0, g], sidx_v.at[slot])
                    pltpu.sync_copy(e_hbm.at[1, g], didx_v.at[slot])
                else:
                    pltpu.async_copy(e_hbm.at[0, g], sidx_v.at[slot],
                                     sem_i.at[slot])
                    pltpu.async_copy(e_hbm.at[1, g], didx_v.at[slot],
                                     sem_i.at[slot])

            @pl.when(g >= EC)
            def _():
                synth_idx(jj, slot)

        def wait_idx(jj, slot):
            g = wid * CHUNKS + jj

            @pl.when(g < EC)
            def _():
                pltpu.make_async_copy(e_hbm.at[0, 0], sidx_v.at[slot],
                                      sem_i.at[slot]).wait()
                pltpu.make_async_copy(e_hbm.at[1, 0], didx_v.at[slot],
                                      sem_i.at[slot]).wait()

        def issue_logit(islot, slot):
            pltpu.async_copy(as_hbm.at[sidx_v.at[islot]], asv_v.at[slot],
                             sem_l.at[slot])
            pltpu.async_copy(ad_hbm.at[didx_v.at[islot]], adv_v.at[slot],
                             sem_l.at[slot])

        def wait_logit(slot):
            pltpu.make_async_copy(as_hbm.at[pl.ds(0, CHUNK)], asv_v.at[slot],
                                  sem_l.at[slot]).wait()
            pltpu.make_async_copy(ad_hbm.at[pl.ds(0, CHUNK)], adv_v.at[slot],
                                  sem_l.at[slot]).wait()

        def issue_rows(islot, slot):
            pltpu.async_copy(xp_hbm.at[sidx_v.at[islot]], rows_v.at[slot],
                             sem_g.at[slot])

        def wait_rows(slot):
            pltpu.make_async_copy(xp_hbm.at[pl.ds(0, CHUNK)], rows_v.at[slot],
                                  sem_g.at[slot]).wait()

        def issue_scatter(islot, slot):
            pltpu.async_copy(rows_v.at[slot], accum_sh.at[didx_v.at[islot]],
                             sem_sr.at[slot], add=True)
            pltpu.async_copy(alpha_v.at[slot], den_sh.at[didx_v.at[islot]],
                             sem_sd.at[slot], add=True)

        def wait_scatter(slot):
            pltpu.make_async_copy(xp_hbm.at[pl.ds(0, CHUNK)], rows_v.at[slot],
                                  sem_sr.at[slot]).wait()
            pltpu.make_async_copy(as_hbm.at[pl.ds(0, CHUNK)], alpha_v.at[slot],
                                  sem_sd.at[slot]).wait()

        prep_idx(0, 0, sync=True)
        prep_idx(1, 1, sync=False)
        issue_logit(0, 0)
        issue_rows(0, 0)

        def chunk_body(j, _):
            b = lax.rem(j, 2)
            nb = 1 - b
            i3 = lax.rem(j, 3)
            i3n = lax.rem(j + 1, 3)
            i3nn = lax.rem(j + 2, 3)

            @pl.when(j >= 1)
            def _():
                wait_scatter(nb)

            @pl.when(j <= CHUNKS - 2)
            def _():
                wait_idx(j + 1, i3n)
                issue_logit(i3n, nb)
                issue_rows(i3n, nb)

            wait_rows(b)
            wait_logit(b)

            @pl.when(j <= CHUNKS - 3)
            def _():
                prep_idx(j + 2, i3nn, sync=False)

            base = (wid * CHUNKS + j) * CHUNK
            for i in range(CHUNK // LANES):
                al = (asv_v[b, pl.ds(i * LANES, LANES)]
                      + adv_v[b, pl.ds(i * LANES, LANES)])
                al = jnp.where(al > 0, al, NEG_SLOPE * al)
                ev = jnp.exp(al)
                eid = base + i * LANES + iota16
                ev = jnp.where(eid < n_edges_real, ev, 0.0)
                alpha_v[b, pl.ds(i * LANES, LANES)] = ev

            def srow(r, _):
                asp = plsc.load_gather(
                    alpha_v,
                    [jnp.broadcast_to(b, (LANES,)).astype(jnp.int32),
                     jnp.broadcast_to(r, (LANES,)).astype(jnp.int32)])
                for c in range(D // LANES):
                    rows_v[b, r, pl.ds(c * LANES, LANES)] = (
                        rows_v[b, r, pl.ds(c * LANES, LANES)] * asp)
                return 0

            lax.fori_loop(0, CHUNK, srow, 0)
            issue_scatter(i3, b)
            return 0

        lax.fori_loop(0, CHUNKS, chunk_body, 0)
        wait_scatter((CHUNKS - 1) % 2)
        plsc.subcore_barrier()

        for off in range(0, rpt, CHUNK):
            cnt = min(CHUNK, rpt - off)
            pltpu.sync_copy(accum_sh.at[pl.ds(sid * rpt + off, cnt)],
                            rows_v.at[0, pl.ds(0, cnt)])
            pltpu.sync_copy(rows_v.at[0, pl.ds(0, cnt)],
                            out_hbm.at[cid, pl.ds(sid * rpt + off, cnt)])
        pltpu.sync_copy(den_sh.at[pl.ds(sid * rpt, rpt)], den_v)
        pltpu.sync_copy(den_v, den_hbm.at[pl.ds(cid * NP + sid * rpt, rpt)])

    mesh = plsc.VectorSubcoreMesh(core_axis_name="c", subcore_axis_name="s")
    return pl.kernel(
        body,
        out_type=[
            jax.ShapeDtypeStruct((NC, NP, D), jnp.float32),
            jax.ShapeDtypeStruct((NC * NP,), jnp.float32),
        ],
        mesh=mesh,
        compiler_params=pltpu.CompilerParams(needs_layout_passes=False),
        scratch_types=[
            pltpu.VMEM((3, CHUNK), jnp.int32),
            pltpu.VMEM((3, CHUNK), jnp.int32),
            pltpu.VMEM((2, CHUNK), jnp.float32),
            pltpu.VMEM((2, CHUNK), jnp.float32),
            pltpu.VMEM((2, CHUNK), jnp.float32),
            pltpu.VMEM((2, CHUNK, D), jnp.float32),
            pltpu.VMEM((rpt,), jnp.float32),
            pltpu.VMEM_SHARED((NP, D), jnp.float32),
            pltpu.VMEM_SHARED((NP,), jnp.float32),
            pltpu.SemaphoreType.DMA((3,)),
            pltpu.SemaphoreType.DMA((2,)),
            pltpu.SemaphoreType.DMA((2,)),
            pltpu.SemaphoreType.DMA((2,)),
            pltpu.SemaphoreType.DMA((2,)),
        ],
    )(xp, a_srcH, a_dstH, e3)


def _tc_combine(accd, dend3, bias2, n_real):
    D = bias2.shape[1]

    def body(acc_ref, den_ref, b_ref, o_ref):
        num = acc_ref[0] + acc_ref[1]
        den = den_ref[0] + den_ref[1]
        o_ref[...] = (num[:n_real] / (den[:n_real] + 1e-16)) + b_ref[...]

    return pl.pallas_call(
        body,
        out_shape=jax.ShapeDtypeStruct((n_real, D), jnp.float32),
    )(accd, dend3, bias2)


def kernel(x, edge_index, W, att_src, att_dst, bias):
    n, d_in = x.shape
    e = edge_index.shape[1]

    NP = n + (-n % NODE_PAD)
    att2 = jnp.stack([att_src, att_dst])
    xp, a2 = _tc_project(x, W, att2, NP)

    ee = e + n
    chunks = -(-ee // (NW * CHUNK))
    e3 = edge_index.reshape(2, e // CHUNK, CHUNK)

    accd, dend = _sc_aggregate(xp, a2[0], a2[1], e3, n, e, ee, chunks)
    dend3 = dend.reshape(NC, NP, 1)
    return _tc_combine(accd, dend3, bias[None, :], n)


# speedup vs baseline: 1.5627x; 1.0078x over previous
import jax
import jax.numpy as jnp
from jax import lax
from jax.experimental import pallas as pl
from jax.experimental.pallas import tpu as pltpu
from jax.experimental.pallas import tpu_sc as plsc

NEG_SLOPE = 0.2
NC, NS, LANES = 2, 16, 16
NW = NC * NS
CHUNK = 128
NODE_PAD = 128


def _tc_project(x, W, att2, NP):
    n, D = x.shape

    def body(x_ref, w_ref, a_ref, xp_ref, as_ref, ad_ref):
        xp = jnp.dot(x_ref[...], w_ref[...], preferred_element_type=jnp.float32)
        xp_ref[:n] = xp
        xp_ref[n:] = jnp.zeros((NP - n, D), jnp.float32)
        a2 = lax.dot_general(
            a_ref[...], xp, (((1,), (1,)), ((), ())),
            preferred_element_type=jnp.float32)
        as_ref[:n] = a2[0]
        as_ref[n:] = jnp.zeros((NP - n,), jnp.float32)
        ad_ref[:n] = a2[1]
        ad_ref[n:] = jnp.zeros((NP - n,), jnp.float32)

    return pl.pallas_call(
        body,
        out_shape=[
            jax.ShapeDtypeStruct((NP, D), jnp.float32),
            jax.ShapeDtypeStruct((NP,), jnp.float32),
            jax.ShapeDtypeStruct((NP,), jnp.float32),
        ],
    )(x, W, att2)


def _sc_aggregate(xp, a_srcH, a_dstH, e3, n_nodes, n_edges, n_edges_real,
                  chunks):
    NP, D = xp.shape
    CHUNKS = chunks
    EC = n_edges // CHUNK
    rpt = NP // NS

    def body(xp_hbm, as_hbm, ad_hbm, e_hbm, out_hbm, den_hbm,
             sidx_v, didx_v, asv_v, adv_v, alpha_v, rows_v, den_v,
             accum_sh, den_sh, sem_i, sem_l, sem_g, sem_sr, sem_sd):
        cid = lax.axis_index("c")
        sid = lax.axis_index("s")
        wid = cid * NS + sid

        zeros16 = jnp.zeros((LANES,), jnp.float32)

        def zrow(r, _):
            for c in range(D // LANES):
                rows_v[0, r, pl.ds(c * LANES, LANES)] = zeros16
            return 0

        lax.fori_loop(0, CHUNK, zrow, 0)
        for i in range(CHUNK // LANES):
            alpha_v[0, pl.ds(i * LANES, LANES)] = zeros16
        for off in range(0, rpt, CHUNK):
            cnt = min(CHUNK, rpt - off)
            pltpu.sync_copy(rows_v.at[0, pl.ds(0, cnt)],
                            accum_sh.at[pl.ds(sid * rpt + off, cnt)])
            pltpu.sync_copy(alpha_v.at[0, pl.ds(0, cnt)],
                            den_sh.at[pl.ds(sid * rpt + off, cnt)])
        plsc.subcore_barrier()

        iota16 = lax.iota(jnp.int32, LANES)

        def synth_idx(jj, slot):
            tbase = (wid * CHUNKS + jj) * CHUNK - n_edges
            for i in range(CHUNK // LANES):
                t = tbase + i * LANES + iota16
                v = jnp.where(t < n_nodes, t, t - n_nodes)
                sidx_v[slot, pl.ds(i * LANES, LANES)] = v
                didx_v[slot, pl.ds(i * LANES, LANES)] = v

        def prep_idx(jj, slot, sync):
            g = wid * CHUNKS + jj

            @pl.when(g < EC)
            def _():
                if sync:
                    pltpu.sync_copy(e_hbm.at[0, g], sidx_v.at[slot])
                    pltpu.sync_copy(e_hbm.at[1, g], didx_v.at[slot])
                else:
                    pltpu.async_copy(e_hbm.at[0, g], sidx_v.at[slot],
                                     sem_i.at[slot])
                    pltpu.async_copy(e_hbm.at[1, g], didx_v.at[slot],
                                     sem_i.at[slot])

            @pl.when(g >= EC)
            def _():
                synth_idx(jj, slot)

        def wait_idx(jj, slot):
            g = wid * CHUNKS + jj

            @pl.when(g < EC)
            def _():
                pltpu.make_async_copy(e_hbm.at[0, 0], sidx_v.at[slot],
                                      sem_i.at[slot]).wait()
                pltpu.make_async_copy(e_hbm.at[1, 0], didx_v.at[slot],
                                      sem_i.at[slot]).wait()

        def issue_logit(islot, slot):
            pltpu.async_copy(as_hbm.at[sidx_v.at[islot]], asv_v.at[slot],
                             sem_l.at[slot])
            pltpu.async_copy(ad_hbm.at[didx_v.at[islot]], adv_v.at[slot],
                             sem_l.at[slot])

        def wait_logit(slot):
            pltpu.make_async_copy(as_hbm.at[pl.ds(0, CHUNK)], asv_v.at[slot],
                                  sem_l.at[slot]).wait()
            pltpu.make_async_copy(ad_hbm.at[pl.ds(0, CHUNK)], adv_v.at[slot],
                                  sem_l.at[slot]).wait()

        def issue_rows(islot, slot):
            pltpu.async_copy(xp_hbm.at[sidx_v.at[islot]], rows_v.at[slot],
                             sem_g.at[slot])

        def wait_rows(slot):
            pltpu.make_async_copy(xp_hbm.at[pl.ds(0, CHUNK)], rows_v.at[slot],
                                  sem_g.at[slot]).wait()

        def issue_scatter(islot, slot):
            pltpu.async_copy(rows_v.at[slot], accum_sh.at[didx_v.at[islot]],
                             sem_sr.at[slot], add=True)
            pltpu.async_copy(alpha_v.at[slot], den_sh.at[didx_v.at[islot]],
                             sem_sd.at[slot], add=True)

        def wait_scatter(slot):
            pltpu.make_async_copy(xp_hbm.at[pl.ds(0, CHUNK)], rows_v.at[slot],
                                  sem_sr.at[slot]).wait()
            pltpu.make_async_copy(as_hbm.at[pl.ds(0, CHUNK)], alpha_v.at[slot],
                                  sem_sd.at[slot]).wait()

        prep_idx(0, 0, sync=True)
        prep_idx(1, 1, sync=False)
        issue_logit(0, 0)
        issue_rows(0, 0)

        def chunk_body(j, _):
            b = lax.rem(j, 2)
            nb = 1 - b
            i3 = lax.rem(j, 3)
            i3n = lax.rem(j + 1, 3)
            i3nn = lax.rem(j + 2, 3)

            @pl.when(j >= 1)
            def _():
                wait_scatter(nb)

            @pl.when(j <= CHUNKS - 2)
            def _():
                wait_idx(j + 1, i3n)
                issue_logit(i3n, nb)
                issue_rows(i3n, nb)

            wait_rows(b)
            wait_logit(b)

            @pl.when(j <= CHUNKS - 3)
            def _():
                prep_idx(j + 2, i3nn, sync=False)

            base = (wid * CHUNKS + j) * CHUNK
            for i in range(CHUNK // LANES):
                al = (asv_v[b, pl.ds(i * LANES, LANES)]
                      + adv_v[b, pl.ds(i * LANES, LANES)])
                al = jnp.where(al > 0, al, NEG_SLOPE * al)
                ev = jnp.exp(al)
                eid = base + i * LANES + iota16
                ev = jnp.where(eid < n_edges_real, ev, 0.0)
                alpha_v[b, pl.ds(i * LANES, LANES)] = ev

            def srow(r, _):
                asp = plsc.load_gather(
                    alpha_v,
                    [jnp.broadcast_to(b, (LANES,)).astype(jnp.int32),
                     jnp.broadcast_to(r, (LANES,)).astype(jnp.int32)])
                for c in range(D // LANES):
                    rows_v[b, r, pl.ds(c * LANES, LANES)] = (
                        rows_v[b, r, pl.ds(c * LANES, LANES)] * asp)
                return 0

            lax.fori_loop(0, CHUNK, srow, 0)
            issue_scatter(i3, b)
            return 0

        lax.fori_loop(0, CHUNKS, chunk_body, 0)
        wait_scatter((CHUNKS - 1) % 2)
        plsc.subcore_barrier()

        for off in range(0, rpt, CHUNK):
            cnt = min(CHUNK, rpt - off)
            pltpu.sync_copy(accum_sh.at[pl.ds(sid * rpt + off, cnt)],
                            rows_v.at[0, pl.ds(0, cnt)])
            pltpu.sync_copy(rows_v.at[0, pl.ds(0, cnt)],
                            out_hbm.at[cid, pl.ds(sid * rpt + off, cnt)])
        pltpu.sync_copy(den_sh.at[pl.ds(sid * rpt, rpt)], den_v)
        pltpu.sync_copy(den_v, den_hbm.at[pl.ds(cid * NP + sid * rpt, rpt)])

    mesh = plsc.VectorSubcoreMesh(core_axis_name="c", subcore_axis_name="s")
    return pl.kernel(
        body,
        out_type=[
            jax.ShapeDtypeStruct((NC, NP, D), jnp.float32),
            jax.ShapeDtypeStruct((NC * NP,), jnp.float32),
        ],
        mesh=mesh,
        compiler_params=pltpu.CompilerParams(needs_layout_passes=False),
        scratch_types=[
            pltpu.VMEM((3, CHUNK), jnp.int32),
            pltpu.VMEM((3, CHUNK), jnp.int32),
            pltpu.VMEM((2, CHUNK), jnp.float32),
            pltpu.VMEM((2, CHUNK), jnp.float32),
            pltpu.VMEM((2, CHUNK), jnp.float32),
            pltpu.VMEM((2, CHUNK, D), jnp.float32),
            pltpu.VMEM((rpt,), jnp.float32),
            pltpu.VMEM_SHARED((NP, D), jnp.float32),
            pltpu.VMEM_SHARED((NP,), jnp.float32),
            pltpu.SemaphoreType.DMA((3,)),
            pltpu.SemaphoreType.DMA((2,)),
            pltpu.SemaphoreType.DMA((2,)),
            pltpu.SemaphoreType.DMA((2,)),
            pltpu.SemaphoreType.DMA((2,)),
        ],
    )(xp, a_srcH, a_dstH, e3)


def _tc_combine(accd, dend3, bias2, n_real):
    D = bias2.shape[1]

    def body(acc_ref, den_ref, b_ref, o_ref):
        num = acc_ref[0] + acc_ref[1]
        den = den_ref[0] + den_ref[1]
        o_ref[...] = (num[:n_real] / (den[:n_real] + 1e-16)) + b_ref[...]

    return pl.pallas_call(
        body,
        out_shape=jax.ShapeDtypeStruct((n_real, D), jnp.float32),
    )(accd, dend3, bias2)


def kernel(x, edge_index, W, att_src, att_dst, bias):
    n, d_in = x.shape
    e = edge_index.shape[1]

    NP = n + (-n % NODE_PAD)
    att2 = jnp.stack([att_src, att_dst])
    xp, a_src, a_dst = _tc_project(x, W, att2, NP)

    ee = e + n
    chunks = -(-ee // (NW * CHUNK))
    e3 = edge_index.reshape(2, e // CHUNK, CHUNK)

    accd, dend = _sc_aggregate(xp, a_src, a_dst, e3, n, e, ee, chunks)
    dend3 = dend.reshape(NC, NP, 1)
    return _tc_combine(accd, dend3, bias[None, :], n)


# speedup vs baseline: 1.5640x; 1.0008x over previous
import jax
import jax.numpy as jnp
from jax import lax
from jax.experimental import pallas as pl
from jax.experimental.pallas import tpu as pltpu
from jax.experimental.pallas import tpu_sc as plsc

NEG_SLOPE = 0.2
NC, NS, LANES = 2, 16, 16
NW = NC * NS
CHUNK = 128
NODE_PAD = 128


def _tc_project(x, W, att2, NP):
    n, D = x.shape

    def body(x_ref, w_ref, a_ref, xp_ref, as_ref, ad_ref):
        xp = jnp.dot(x_ref[...], w_ref[...], preferred_element_type=jnp.float32)
        xp_ref[:n] = xp
        xp_ref[n:] = jnp.zeros((NP - n, D), jnp.float32)
        a2 = lax.dot_general(
            a_ref[...], xp, (((1,), (1,)), ((), ())),
            preferred_element_type=jnp.float32)
        as_ref[:n] = a2[0]
        as_ref[n:] = jnp.zeros((NP - n,), jnp.float32)
        ad_ref[:n] = a2[1]
        ad_ref[n:] = jnp.zeros((NP - n,), jnp.float32)

    return pl.pallas_call(
        body,
        out_shape=[
            jax.ShapeDtypeStruct((NP, D), jnp.float32),
            jax.ShapeDtypeStruct((NP,), jnp.float32),
            jax.ShapeDtypeStruct((NP,), jnp.float32),
        ],
    )(x, W, att2)


def _sc_aggregate(xp, a_srcH, a_dstH, e3, n_nodes, n_edges, n_edges_real,
                  chunks):
    NP, D = xp.shape
    CHUNKS = chunks
    EC = n_edges // CHUNK
    rpt = NP // NS

    def body(xp_hbm, as_hbm, ad_hbm, e_hbm, out_hbm, den_hbm,
             sidx_v, didx_v, asv_v, adv_v, alpha_v, rows_v, den_v,
             accum_sh, den_sh, sem_i, sem_l, sem_g, sem_sr, sem_sd):
        cid = lax.axis_index("c")
        sid = lax.axis_index("s")
        wid = cid * NS + sid

        zeros16 = jnp.zeros((LANES,), jnp.float32)

        def zrow(r, _):
            for c in range(D // LANES):
                rows_v[0, r, pl.ds(c * LANES, LANES)] = zeros16
            return 0

        lax.fori_loop(0, CHUNK, zrow, 0)
        for i in range(CHUNK // LANES):
            alpha_v[0, pl.ds(i * LANES, LANES)] = zeros16
        for off in range(0, rpt, CHUNK):
            cnt = min(CHUNK, rpt - off)
            pltpu.sync_copy(rows_v.at[0, pl.ds(0, cnt)],
                            accum_sh.at[pl.ds(sid * rpt + off, cnt)])
            pltpu.sync_copy(alpha_v.at[0, pl.ds(0, cnt)],
                            den_sh.at[pl.ds(sid * rpt + off, cnt)])
        plsc.subcore_barrier()

        iota16 = lax.iota(jnp.int32, LANES)

        def synth_idx(jj, slot):
            tbase = (wid * CHUNKS + jj) * CHUNK - n_edges
            for i in range(CHUNK // LANES):
                t = tbase + i * LANES + iota16
                v = jnp.where(t < n_nodes, t, t - n_nodes)
                sidx_v[slot, pl.ds(i * LANES, LANES)] = v
                didx_v[slot, pl.ds(i * LANES, LANES)] = v

        def prep_idx(jj, slot, sync):
            g = wid * CHUNKS + jj

            @pl.when(g < EC)
            def _():
                if sync:
                    pltpu.sync_copy(e_hbm.at[0, g], sidx_v.at[slot])
                    pltpu.sync_copy(e_hbm.at[1, g], didx_v.at[slot])
                else:
                    pltpu.async_copy(e_hbm.at[0, g], sidx_v.at[slot],
                                     sem_i.at[slot])
                    pltpu.async_copy(e_hbm.at[1, g], didx_v.at[slot],
                                     sem_i.at[slot])

            @pl.when(g >= EC)
            def _():
                synth_idx(jj, slot)

        def wait_idx(jj, slot):
            g = wid * CHUNKS + jj

            @pl.when(g < EC)
            def _():
                pltpu.make_async_copy(e_hbm.at[0, 0], sidx_v.at[slot],
                                      sem_i.at[slot]).wait()
                pltpu.make_async_copy(e_hbm.at[1, 0], didx_v.at[slot],
                                      sem_i.at[slot]).wait()

        def issue_logit(islot, slot):
            pltpu.async_copy(as_hbm.at[sidx_v.at[islot]], asv_v.at[slot],
                             sem_l.at[slot])
            pltpu.async_copy(ad_hbm.at[didx_v.at[islot]], adv_v.at[slot],
                             sem_l.at[slot])

        def wait_logit(slot):
            pltpu.make_async_copy(as_hbm.at[pl.ds(0, CHUNK)], asv_v.at[slot],
                                  sem_l.at[slot]).wait()
            pltpu.make_async_copy(ad_hbm.at[pl.ds(0, CHUNK)], adv_v.at[slot],
                                  sem_l.at[slot]).wait()

        def issue_rows(islot, slot):
            pltpu.async_copy(xp_hbm.at[sidx_v.at[islot]], rows_v.at[slot],
                             sem_g.at[slot])

        def wait_rows(slot):
            pltpu.make_async_copy(xp_hbm.at[pl.ds(0, CHUNK)], rows_v.at[slot],
                                  sem_g.at[slot]).wait()

        def issue_scatter(islot, slot):
            pltpu.async_copy(rows_v.at[slot], accum_sh.at[didx_v.at[islot]],
                             sem_sr.at[slot], add=True)
            pltpu.async_copy(alpha_v.at[slot], den_sh.at[didx_v.at[islot]],
                             sem_sd.at[slot], add=True)

        def wait_scatter(slot):
            pltpu.make_async_copy(xp_hbm.at[pl.ds(0, CHUNK)], rows_v.at[slot],
                                  sem_sr.at[slot]).wait()
            pltpu.make_async_copy(as_hbm.at[pl.ds(0, CHUNK)], alpha_v.at[slot],
                                  sem_sd.at[slot]).wait()

        prep_idx(0, 0, sync=True)
        prep_idx(1, 1, sync=False)
        issue_logit(0, 0)
        issue_rows(0, 0)

        def chunk_body(j, _):
            b = lax.rem(j, 2)
            nb = 1 - b
            i3 = lax.rem(j, 3)
            i3n = lax.rem(j + 1, 3)
            i3nn = lax.rem(j + 2, 3)

            @pl.when(j >= 1)
            def _():
                wait_scatter(nb)

            @pl.when(j <= CHUNKS - 2)
            def _():
                wait_idx(j + 1, i3n)
                issue_logit(i3n, nb)
                issue_rows(i3n, nb)

            wait_rows(b)
            wait_logit(b)

            @pl.when(j <= CHUNKS - 3)
            def _():
                prep_idx(j + 2, i3nn, sync=False)

            base = (wid * CHUNKS + j) * CHUNK
            for i in range(CHUNK // LANES):
                al = (asv_v[b, pl.ds(i * LANES, LANES)]
                      + adv_v[b, pl.ds(i * LANES, LANES)])
                al = jnp.where(al > 0, al, NEG_SLOPE * al)
                ev = jnp.exp(al)
                eid = base + i * LANES + iota16
                ev = jnp.where(eid < n_edges_real, ev, 0.0)
                alpha_v[b, pl.ds(i * LANES, LANES)] = ev

            def srow(r, _):
                asp = plsc.load_gather(
                    alpha_v,
                    [jnp.broadcast_to(b, (LANES,)).astype(jnp.int32),
                     jnp.broadcast_to(r, (LANES,)).astype(jnp.int32)])
                for c in range(D // LANES):
                    rows_v[b, r, pl.ds(c * LANES, LANES)] = (
                        rows_v[b, r, pl.ds(c * LANES, LANES)] * asp)
                return 0

            lax.fori_loop(0, CHUNK, srow, 0)
            issue_scatter(i3, b)
            return 0

        lax.fori_loop(0, CHUNKS, chunk_body, 0)
        wait_scatter((CHUNKS - 1) % 2)
        plsc.subcore_barrier()

        pltpu.sync_copy(accum_sh.at[pl.ds(sid * rpt, rpt)],
                        out_hbm.at[cid, pl.ds(sid * rpt, rpt)])
        pltpu.sync_copy(den_sh.at[pl.ds(sid * rpt, rpt)], den_v)
        pltpu.sync_copy(den_v, den_hbm.at[pl.ds(cid * NP + sid * rpt, rpt)])

    mesh = plsc.VectorSubcoreMesh(core_axis_name="c", subcore_axis_name="s")
    return pl.kernel(
        body,
        out_type=[
            jax.ShapeDtypeStruct((NC, NP, D), jnp.float32),
            jax.ShapeDtypeStruct((NC * NP,), jnp.float32),
        ],
        mesh=mesh,
        compiler_params=pltpu.CompilerParams(needs_layout_passes=False),
        scratch_types=[
            pltpu.VMEM((3, CHUNK), jnp.int32),
            pltpu.VMEM((3, CHUNK), jnp.int32),
            pltpu.VMEM((2, CHUNK), jnp.float32),
            pltpu.VMEM((2, CHUNK), jnp.float32),
            pltpu.VMEM((2, CHUNK), jnp.float32),
            pltpu.VMEM((2, CHUNK, D), jnp.float32),
            pltpu.VMEM((rpt,), jnp.float32),
            pltpu.VMEM_SHARED((NP, D), jnp.float32),
            pltpu.VMEM_SHARED((NP,), jnp.float32),
            pltpu.SemaphoreType.DMA((3,)),
            pltpu.SemaphoreType.DMA((2,)),
            pltpu.SemaphoreType.DMA((2,)),
            pltpu.SemaphoreType.DMA((2,)),
            pltpu.SemaphoreType.DMA((2,)),
        ],
    )(xp, a_srcH, a_dstH, e3)


def _tc_combine(accd, dend3, bias2, n_real):
    D = bias2.shape[1]

    def body(acc_ref, den_ref, b_ref, o_ref):
        num = acc_ref[0] + acc_ref[1]
        den = den_ref[0] + den_ref[1]
        o_ref[...] = (num[:n_real] / (den[:n_real] + 1e-16)) + b_ref[...]

    return pl.pallas_call(
        body,
        out_shape=jax.ShapeDtypeStruct((n_real, D), jnp.float32),
    )(accd, dend3, bias2)


def kernel(x, edge_index, W, att_src, att_dst, bias):
    n, d_in = x.shape
    e = edge_index.shape[1]

    NP = n + (-n % NODE_PAD)
    att2 = jnp.stack([att_src, att_dst])
    xp, a_src, a_dst = _tc_project(x, W, att2, NP)

    ee = e + n
    chunks = -(-ee // (NW * CHUNK))
    e3 = edge_index.reshape(2, e // CHUNK, CHUNK)

    accd, dend = _sc_aggregate(xp, a_src, a_dst, e3, n, e, ee, chunks)
    dend3 = dend.reshape(NC, NP, 1)
    return _tc_combine(accd, dend3, bias[None, :], n)
